# Initial kernel scaffold; baseline (speedup 1.0000x reference)
import jax
import jax.numpy as jnp
from jax.experimental import pallas as pl


def kernel(x, sn, node, W1, b1, W2, b2, W3, b3, W4, b4, W5, b5, Wk1, bk1, Wk2, bk2, Wk3, bk3, Wa1, ba1, Wa2, ba2, Wm1, bm1, Wm2, bm2, Wm3, bm3):
    raise NotImplementedError("write your pallas kernel here")


# speedup vs baseline: 20.7785x; 20.7785x over previous
import functools

import jax
import jax.numpy as jnp
from jax import lax
from jax.experimental import pallas as pl
from jax.experimental.pallas import tpu as pltpu
from jax.experimental.pallas import tpu_sc as plsc

_B, _N, _M = 4, 16384, 512
_BIG = 3.0e38


def _topk_body(k, n_cols, row_off_stride, a_ref, b_ref, out_ref, dist_s):
    mb = a_ref.shape[2]
    av = a_ref[0]
    bv = b_ref[0]
    a2 = jnp.sum(av * av, axis=0)
    b2 = jnp.sum(bv * bv, axis=0)
    cross = lax.dot_general(
        av, bv, dimension_numbers=(((0,), (0,)), ((), ())),
        preferred_element_type=jnp.float32, precision=lax.Precision.HIGHEST)
    d = jnp.maximum(a2[:, None] + b2[None, :] - 2.0 * cross, 0.0)
    dist_s[...] = d
    boff = pl.program_id(0) * row_off_stride
    iota = lax.broadcasted_iota(jnp.int32, (mb, n_cols), 1)

    def body(j, carry):
        dd = dist_s[...]
        mn = jnp.min(dd, axis=1)
        am = jnp.min(jnp.where(dd <= mn[:, None], iota, n_cols), axis=1)
        out_ref[0, pl.ds(j, 1), :] = (am + boff)[None, :]
        dist_s[...] = jnp.where(iota == am[:, None], _BIG, dd)
        return carry

    lax.fori_loop(0, k, body, 0)


def _topk(a, b, k, mb):
    B, _, M = a.shape
    N = b.shape[2]
    fn = functools.partial(_topk_body, k, N, N)
    return pl.pallas_call(
        fn,
        grid=(B, M // mb),
        in_specs=[
            pl.BlockSpec((1, 3, mb), lambda bi, mi: (bi, 0, mi)),
            pl.BlockSpec((1, 3, N), lambda bi, mi: (bi, 0, 0)),
        ],
        out_specs=pl.BlockSpec((1, k, mb), lambda bi, mi: (bi, 0, mi)),
        out_shape=jax.ShapeDtypeStruct((B, k, M), jnp.int32),
        scratch_shapes=[pltpu.VMEM((mb, N), jnp.float32)],
    )(a, b)


def _sc_gather(table, idx):
    V, D = table.shape
    Bt = idx.shape[0]
    info = plsc.get_sparse_core_info()
    nc, ns = info.num_cores, info.num_subcores
    nw = nc * ns
    bpw = Bt // nw
    rows = bpw
    while rows * D * 4 > 262144:
        rows //= 2
    nch = bpw // rows
    mesh = plsc.VectorSubcoreMesh(core_axis_name="c", subcore_axis_name="s")

    @functools.partial(
        pl.kernel, mesh=mesh,
        out_type=jax.ShapeDtypeStruct((Bt, D), jnp.float32),
        scratch_types=[
            pltpu.VMEM((rows,), jnp.int32),
            pltpu.VMEM((rows, D), jnp.float32),
            pltpu.SemaphoreType.DMA,
        ],
    )
    def k(tab_hbm, idx_hbm, out_hbm, idx_v, rows_v, sem):
        wid = lax.axis_index("s") * nc + lax.axis_index("c")
        base = wid * bpw
        for c in range(nch):
            off = base + c * rows
            pltpu.sync_copy(idx_hbm.at[pl.ds(off, rows)], idx_v)
            pltpu.async_copy(tab_hbm.at[idx_v], rows_v, sem).wait()
            pltpu.sync_copy(rows_v, out_hbm.at[pl.ds(off, rows)])

    return k(table, idx)


def _mm(x, w, b):
    return lax.dot_general(
        x, w, dimension_numbers=(((1,), (0,)), ((), ())),
        preferred_element_type=jnp.float32,
        precision=lax.Precision.HIGHEST) + b


def _stage1_body(g_ref, nd_ref, w1, b1, w2, b2, w3, b3, w4, b4, w5, b5,
                 out_ref):
    mb, K, Dp = g_ref.shape[1], g_ref.shape[2], g_ref.shape[3]
    g = g_ref[0]
    nd = nd_ref[0]
    rel = g[..., 0:3] - nd[:, None, :]
    inp = jnp.concatenate([rel, g[..., 3:6]], axis=-1).reshape(mb * K, 6)
    h = jnp.maximum(_mm(inp, w1[...], b1[...]), 0.0)
    h = jnp.maximum(_mm(h, w2[...], b2[...]), 0.0)
    h = jnp.maximum(_mm(h, w3[...], b3[...]), 0.0)
    h3 = h.reshape(mb, K, 64)
    hm = jnp.max(h3, axis=1)
    cat = jnp.concatenate(
        [h3, jnp.broadcast_to(hm[:, None, :], h3.shape)], axis=-1
    ).reshape(mb * K, 128)
    h = jnp.maximum(_mm(cat, w4[...], b4[...]), 0.0)
    h = jnp.maximum(_mm(h, w5[...], b5[...]), 0.0)
    out_ref[0] = jnp.max(h.reshape(mb, K, 128), axis=1)


def _stage1(g, node_t, w1, b1, w2, b2, w3, b3, w4, b4, w5, b5, mb=64):
    B, M, K, Dp = g.shape
    full = lambda s: pl.BlockSpec(s, lambda bi, mi: tuple(0 for _ in s))
    return pl.pallas_call(
        _stage1_body,
        grid=(B, M // mb),
        in_specs=[
            pl.BlockSpec((1, mb, K, Dp), lambda bi, mi: (bi, mi, 0, 0)),
            pl.BlockSpec((1, mb, 3), lambda bi, mi: (bi, mi, 0)),
            full(w1.shape), full(b1.shape), full(w2.shape), full(b2.shape),
            full(w3.shape), full(b3.shape), full(w4.shape), full(b4.shape),
            full(w5.shape), full(b5.shape),
        ],
        out_specs=pl.BlockSpec((1, mb, 128), lambda bi, mi: (bi, mi, 0)),
        out_shape=jax.ShapeDtypeStruct((B, M, 128), jnp.float32),
    )(g, node_t, w1, b1, w2, b2, w3, b3, w4, b4, w5, b5)


def _stage2_body(g_ref, nd_ref, wk1, bk1, wk2, bk2, wk3, bk3, wa1, ba1,
                 wa2, ba2, out_ref):
    mb, K = g_ref.shape[1], g_ref.shape[2]
    g = g_ref[0]
    nd = nd_ref[0]
    rel = g[..., 0:3] - nd[:, None, :]
    inp = jnp.concatenate([rel, g[..., 3:131]], axis=-1).reshape(mb * K, 131)
    h = jnp.maximum(_mm(inp, wk1[...], bk1[...]), 0.0)
    h = jnp.maximum(_mm(h, wk2[...], bk2[...]), 0.0)
    h = jnp.maximum(_mm(h, wk3[...], bk3[...]), 0.0)
    h3 = h.reshape(mb, K, 256)
    hm = jnp.max(h3, axis=1)
    cat = jnp.concatenate(
        [h3, jnp.broadcast_to(hm[:, None, :], h3.shape)], axis=-1
    ).reshape(mb * K, 512)
    h = jnp.maximum(_mm(cat, wa1[...], ba1[...]), 0.0)
    h = jnp.maximum(_mm(h, wa2[...], ba2[...]), 0.0)
    out_ref[0] = jnp.max(h.reshape(mb, K, 512), axis=1)


def _stage2(g, node_t, wk1, bk1, wk2, bk2, wk3, bk3, wa1, ba1, wa2, ba2,
            mb=64):
    B, M, K, Dp = g.shape
    full = lambda s: pl.BlockSpec(s, lambda bi, mi: tuple(0 for _ in s))
    return pl.pallas_call(
        _stage2_body,
        grid=(B, M // mb),
        in_specs=[
            pl.BlockSpec((1, mb, K, Dp), lambda bi, mi: (bi, mi, 0, 0)),
            pl.BlockSpec((1, mb, 3), lambda bi, mi: (bi, mi, 0)),
            full(wk1.shape), full(bk1.shape), full(wk2.shape),
            full(bk2.shape), full(wk3.shape), full(bk3.shape),
            full(wa1.shape), full(ba1.shape), full(wa2.shape),
            full(ba2.shape),
        ],
        out_specs=pl.BlockSpec((1, mb, 512), lambda bi, mi: (bi, mi, 0)),
        out_shape=jax.ShapeDtypeStruct((B, M, 512), jnp.float32),
    )(g, node_t, wk1, bk1, wk2, bk2, wk3, bk3, wa1, ba1, wa2, ba2)


def _final_body(spn_ref, knn_ref, nd_ref, wm1, bm1, wm2, bm2, wm3, bm3,
                kp_ref, sg_ref):
    feat = jnp.concatenate([spn_ref[0], knn_ref[0]], axis=-1)
    y = jnp.maximum(_mm(feat, wm1[...], bm1[...]), 0.0)
    pd = jnp.maximum(_mm(y, wm2[...], bm2[...]), 0.0)
    ks = _mm(pd, wm3[...], bm3[...])
    kp_ref[0] = ks[:, 0:3] + nd_ref[0]
    s = ks[:, 3:4]
    sg_ref[0] = jnp.maximum(s, 0.0) + jnp.log1p(jnp.exp(-jnp.abs(s))) + 0.001


def _final(spn, knn1, node_t, wm1, bm1, wm2, bm2, wm3, bm3):
    B, M, _ = spn.shape
    full = lambda s: pl.BlockSpec(s, lambda bi: tuple(0 for _ in s))
    return pl.pallas_call(
        _final_body,
        grid=(B,),
        in_specs=[
            pl.BlockSpec((1, M, 128), lambda bi: (bi, 0, 0)),
            pl.BlockSpec((1, M, 512), lambda bi: (bi, 0, 0)),
            pl.BlockSpec((1, M, 3), lambda bi: (bi, 0, 0)),
            full(wm1.shape), full(bm1.shape), full(wm2.shape),
            full(bm2.shape), full(wm3.shape), full(bm3.shape),
        ],
        out_specs=[
            pl.BlockSpec((1, M, 3), lambda bi: (bi, 0, 0)),
            pl.BlockSpec((1, M, 1), lambda bi: (bi, 0, 0)),
        ],
        out_shape=[
            jax.ShapeDtypeStruct((B, M, 3), jnp.float32),
            jax.ShapeDtypeStruct((B, M, 1), jnp.float32),
        ],
    )(spn, knn1, node_t, wm1, bm1, wm2, bm2, wm3, bm3)


def kernel(x, sn, node, W1, b1, W2, b2, W3, b3, W4, b4, W5, b5, Wk1, bk1,
           Wk2, bk2, Wk3, bk3, Wa1, ba1, Wa2, ba2, Wm1, bm1, Wm2, bm2,
           Wm3, bm3):
    B, _, N = x.shape
    M = node.shape[2]
    node_t = jnp.transpose(node, (0, 2, 1))

    idx1 = _topk(node, x, k=64, mb=128)
    tab1 = jnp.transpose(jnp.concatenate([x, sn], axis=1), (0, 2, 1))
    tab1 = jnp.pad(tab1.reshape(B * N, 6), ((0, 0), (0, 122)))
    flat1 = jnp.transpose(idx1, (0, 2, 1)).reshape(B * M * 64)
    g1 = _sc_gather(tab1, flat1).reshape(B, M, 64, 128)
    spn = _stage1(g1, node_t, W1.T, b1[None, :], W2.T, b2[None, :],
                  W3.T, b3[None, :], W4.T, b4[None, :], W5.T, b5[None, :])

    idx2 = _topk(node, node, k=16, mb=128)
    tab2 = jnp.concatenate([node_t.reshape(B * M, 3),
                            spn.reshape(B * M, 128)], axis=1)
    tab2 = jnp.pad(tab2, ((0, 0), (0, 125)))
    flat2 = jnp.transpose(idx2, (0, 2, 1)).reshape(B * M * 16)
    g2 = _sc_gather(tab2, flat2).reshape(B, M, 16, 256)
    knn1 = _stage2(g2, node_t, Wk1.T, bk1[None, :], Wk2.T, bk2[None, :],
                   Wk3.T, bk3[None, :], Wa1.T, ba1[None, :],
                   Wa2.T, ba2[None, :])

    kp, sg = _final(spn, knn1, node_t, Wm1.T, bm1[None, :],
                    Wm2.T, bm2[None, :], Wm3.T, bm3[None, :])
    keypoints = jnp.transpose(kp, (0, 2, 1))
    sigmas = sg.reshape(B, M)
    return node, keypoints, sigmas


# speedup vs baseline: 22.4304x; 1.0795x over previous
import functools

import jax
import jax.numpy as jnp
from jax import lax
from jax.experimental import pallas as pl
from jax.experimental.pallas import tpu as pltpu
from jax.experimental.pallas import tpu_sc as plsc

_B, _N, _M = 4, 16384, 512
_BIG = 3.0e38


def _topk_body(k, n_cols, row_off_stride, a_ref, b_ref, out_ref, dist_s):
    mb = a_ref.shape[2]
    av = a_ref[0]
    bv = b_ref[0]
    a2 = jnp.sum(av * av, axis=0)
    b2 = jnp.sum(bv * bv, axis=0)
    cross = lax.dot_general(
        av, bv, dimension_numbers=(((0,), (0,)), ((), ())),
        preferred_element_type=jnp.float32, precision=lax.Precision.HIGHEST)
    d = jnp.maximum(a2[:, None] + b2[None, :] - 2.0 * cross, 0.0)
    boff = pl.program_id(0) * row_off_stride
    iota = lax.broadcasted_iota(jnp.int32, (mb, n_cols), 1)
    E = 8 if k % 8 == 0 else 4

    def extract(e_base, dd):
        for e in range(E):
            mn = jnp.min(dd, axis=1)
            am = jnp.min(jnp.where(dd <= mn[:, None], iota, n_cols), axis=1)
            out_ref[0, pl.ds(e_base + e, 1), :] = (am + boff)[None, :]
            dd = jnp.where(iota == am[:, None], _BIG, dd)
        return dd

    dist_s[...] = extract(0, d)

    def body(j, carry):
        dist_s[...] = extract(j * E, dist_s[...])
        return carry

    lax.fori_loop(1, k // E, body, 0)


def _topk(a, b, k, mb):
    B, _, M = a.shape
    N = b.shape[2]
    fn = functools.partial(_topk_body, k, N, N)
    return pl.pallas_call(
        fn,
        grid=(B, M // mb),
        in_specs=[
            pl.BlockSpec((1, 3, mb), lambda bi, mi: (bi, 0, mi)),
            pl.BlockSpec((1, 3, N), lambda bi, mi: (bi, 0, 0)),
        ],
        out_specs=pl.BlockSpec((1, k, mb), lambda bi, mi: (bi, 0, mi)),
        out_shape=jax.ShapeDtypeStruct((B, k, M), jnp.int32),
        scratch_shapes=[pltpu.VMEM((mb, N), jnp.float32)],
    )(a, b)


def _sc_gather(table, idx):
    V, D = table.shape
    Bt = idx.shape[0]
    info = plsc.get_sparse_core_info()
    nc, ns = info.num_cores, info.num_subcores
    nw = nc * ns
    bpw = Bt // nw
    rows = bpw
    while rows * D * 4 > 262144:
        rows //= 2
    nch = bpw // rows
    mesh = plsc.VectorSubcoreMesh(core_axis_name="c", subcore_axis_name="s")

    @functools.partial(
        pl.kernel, mesh=mesh,
        out_type=jax.ShapeDtypeStruct((Bt, D), jnp.float32),
        scratch_types=[
            pltpu.VMEM((rows,), jnp.int32),
            pltpu.VMEM((rows, D), jnp.float32),
            pltpu.SemaphoreType.DMA,
        ],
    )
    def k(tab_hbm, idx_hbm, out_hbm, idx_v, rows_v, sem):
        wid = lax.axis_index("s") * nc + lax.axis_index("c")
        base = wid * bpw
        for c in range(nch):
            off = base + c * rows
            pltpu.sync_copy(idx_hbm.at[pl.ds(off, rows)], idx_v)
            pltpu.async_copy(tab_hbm.at[idx_v], rows_v, sem).wait()
            pltpu.sync_copy(rows_v, out_hbm.at[pl.ds(off, rows)])

    return k(table, idx)


def _mm(x, w, b):
    return lax.dot_general(
        x, w, dimension_numbers=(((1,), (0,)), ((), ())),
        preferred_element_type=jnp.float32,
        precision=lax.Precision.HIGHEST) + b


def _stage1_body(g_ref, nd_ref, w1, b1, w2, b2, w3, b3, w4, b4, w5, b5,
                 out_ref):
    mb, K, Dp = g_ref.shape[1], g_ref.shape[2], g_ref.shape[3]
    g = g_ref[0]
    nd = nd_ref[0]
    rel = g[..., 0:3] - nd[:, None, :]
    inp = jnp.concatenate([rel, g[..., 3:6]], axis=-1).reshape(mb * K, 6)
    h = jnp.maximum(_mm(inp, w1[...], b1[...]), 0.0)
    h = jnp.maximum(_mm(h, w2[...], b2[...]), 0.0)
    h = jnp.maximum(_mm(h, w3[...], b3[...]), 0.0)
    h3 = h.reshape(mb, K, 64)
    hm = jnp.max(h3, axis=1)
    cat = jnp.concatenate(
        [h3, jnp.broadcast_to(hm[:, None, :], h3.shape)], axis=-1
    ).reshape(mb * K, 128)
    h = jnp.maximum(_mm(cat, w4[...], b4[...]), 0.0)
    h = jnp.maximum(_mm(h, w5[...], b5[...]), 0.0)
    out_ref[0] = jnp.max(h.reshape(mb, K, 128), axis=1)


def _stage1(g, node_t, w1, b1, w2, b2, w3, b3, w4, b4, w5, b5, mb=64):
    B, M, K, Dp = g.shape
    full = lambda s: pl.BlockSpec(s, lambda bi, mi: tuple(0 for _ in s))
    return pl.pallas_call(
        _stage1_body,
        grid=(B, M // mb),
        in_specs=[
            pl.BlockSpec((1, mb, K, Dp), lambda bi, mi: (bi, mi, 0, 0)),
            pl.BlockSpec((1, mb, 3), lambda bi, mi: (bi, mi, 0)),
            full(w1.shape), full(b1.shape), full(w2.shape), full(b2.shape),
            full(w3.shape), full(b3.shape), full(w4.shape), full(b4.shape),
            full(w5.shape), full(b5.shape),
        ],
        out_specs=pl.BlockSpec((1, mb, 128), lambda bi, mi: (bi, mi, 0)),
        out_shape=jax.ShapeDtypeStruct((B, M, 128), jnp.float32),
    )(g, node_t, w1, b1, w2, b2, w3, b3, w4, b4, w5, b5)


def _stage2_body(g_ref, nd_ref, wk1, bk1, wk2, bk2, wk3, bk3, wa1, ba1,
                 wa2, ba2, out_ref):
    mb, K = g_ref.shape[1], g_ref.shape[2]
    g = g_ref[0]
    nd = nd_ref[0]
    rel = g[..., 0:3] - nd[:, None, :]
    inp = jnp.concatenate([rel, g[..., 3:131]], axis=-1).reshape(mb * K, 131)
    h = jnp.maximum(_mm(inp, wk1[...], bk1[...]), 0.0)
    h = jnp.maximum(_mm(h, wk2[...], bk2[...]), 0.0)
    h = jnp.maximum(_mm(h, wk3[...], bk3[...]), 0.0)
    h3 = h.reshape(mb, K, 256)
    hm = jnp.max(h3, axis=1)
    cat = jnp.concatenate(
        [h3, jnp.broadcast_to(hm[:, None, :], h3.shape)], axis=-1
    ).reshape(mb * K, 512)
    h = jnp.maximum(_mm(cat, wa1[...], ba1[...]), 0.0)
    h = jnp.maximum(_mm(h, wa2[...], ba2[...]), 0.0)
    out_ref[0] = jnp.max(h.reshape(mb, K, 512), axis=1)


def _stage2(g, node_t, wk1, bk1, wk2, bk2, wk3, bk3, wa1, ba1, wa2, ba2,
            mb=64):
    B, M, K, Dp = g.shape
    full = lambda s: pl.BlockSpec(s, lambda bi, mi: tuple(0 for _ in s))
    return pl.pallas_call(
        _stage2_body,
        grid=(B, M // mb),
        in_specs=[
            pl.BlockSpec((1, mb, K, Dp), lambda bi, mi: (bi, mi, 0, 0)),
            pl.BlockSpec((1, mb, 3), lambda bi, mi: (bi, mi, 0)),
            full(wk1.shape), full(bk1.shape), full(wk2.shape),
            full(bk2.shape), full(wk3.shape), full(bk3.shape),
            full(wa1.shape), full(ba1.shape), full(wa2.shape),
            full(ba2.shape),
        ],
        out_specs=pl.BlockSpec((1, mb, 512), lambda bi, mi: (bi, mi, 0)),
        out_shape=jax.ShapeDtypeStruct((B, M, 512), jnp.float32),
    )(g, node_t, wk1, bk1, wk2, bk2, wk3, bk3, wa1, ba1, wa2, ba2)


def _final_body(spn_ref, knn_ref, nd_ref, wm1, bm1, wm2, bm2, wm3, bm3,
                kp_ref, sg_ref):
    feat = jnp.concatenate([spn_ref[0], knn_ref[0]], axis=-1)
    y = jnp.maximum(_mm(feat, wm1[...], bm1[...]), 0.0)
    pd = jnp.maximum(_mm(y, wm2[...], bm2[...]), 0.0)
    ks = _mm(pd, wm3[...], bm3[...])
    kp_ref[0] = ks[:, 0:3] + nd_ref[0]
    s = ks[:, 3:4]
    sg_ref[0] = jnp.maximum(s, 0.0) + jnp.log1p(jnp.exp(-jnp.abs(s))) + 0.001


def _final(spn, knn1, node_t, wm1, bm1, wm2, bm2, wm3, bm3):
    B, M, _ = spn.shape
    full = lambda s: pl.BlockSpec(s, lambda bi: tuple(0 for _ in s))
    return pl.pallas_call(
        _final_body,
        grid=(B,),
        in_specs=[
            pl.BlockSpec((1, M, 128), lambda bi: (bi, 0, 0)),
            pl.BlockSpec((1, M, 512), lambda bi: (bi, 0, 0)),
            pl.BlockSpec((1, M, 3), lambda bi: (bi, 0, 0)),
            full(wm1.shape), full(bm1.shape), full(wm2.shape),
            full(bm2.shape), full(wm3.shape), full(bm3.shape),
        ],
        out_specs=[
            pl.BlockSpec((1, M, 3), lambda bi: (bi, 0, 0)),
            pl.BlockSpec((1, M, 1), lambda bi: (bi, 0, 0)),
        ],
        out_shape=[
            jax.ShapeDtypeStruct((B, M, 3), jnp.float32),
            jax.ShapeDtypeStruct((B, M, 1), jnp.float32),
        ],
    )(spn, knn1, node_t, wm1, bm1, wm2, bm2, wm3, bm3)


def kernel(x, sn, node, W1, b1, W2, b2, W3, b3, W4, b4, W5, b5, Wk1, bk1,
           Wk2, bk2, Wk3, bk3, Wa1, ba1, Wa2, ba2, Wm1, bm1, Wm2, bm2,
           Wm3, bm3):
    B, _, N = x.shape
    M = node.shape[2]
    node_t = jnp.transpose(node, (0, 2, 1))

    idx1 = _topk(node, x, k=64, mb=128)
    tab1 = jnp.transpose(jnp.concatenate([x, sn], axis=1), (0, 2, 1))
    tab1 = jnp.pad(tab1.reshape(B * N, 6), ((0, 0), (0, 122)))
    flat1 = jnp.transpose(idx1, (0, 2, 1)).reshape(B * M * 64)
    g1 = _sc_gather(tab1, flat1).reshape(B, M, 64, 128)
    spn = _stage1(g1, node_t, W1.T, b1[None, :], W2.T, b2[None, :],
                  W3.T, b3[None, :], W4.T, b4[None, :], W5.T, b5[None, :])

    idx2 = _topk(node, node, k=16, mb=128)
    tab2 = jnp.concatenate([node_t.reshape(B * M, 3),
                            spn.reshape(B * M, 128)], axis=1)
    tab2 = jnp.pad(tab2, ((0, 0), (0, 125)))
    flat2 = jnp.transpose(idx2, (0, 2, 1)).reshape(B * M * 16)
    g2 = _sc_gather(tab2, flat2).reshape(B, M, 16, 256)
    knn1 = _stage2(g2, node_t, Wk1.T, bk1[None, :], Wk2.T, bk2[None, :],
                   Wk3.T, bk3[None, :], Wa1.T, ba1[None, :],
                   Wa2.T, ba2[None, :])

    kp, sg = _final(spn, knn1, node_t, Wm1.T, bm1[None, :],
                    Wm2.T, bm2[None, :], Wm3.T, bm3[None, :])
    keypoints = jnp.transpose(kp, (0, 2, 1))
    sigmas = sg.reshape(B, M)
    return node, keypoints, sigmas


# speedup vs baseline: 28.0216x; 1.2493x over previous
import functools

import jax
import jax.numpy as jnp
from jax import lax
from jax.experimental import pallas as pl
from jax.experimental.pallas import tpu as pltpu
from jax.experimental.pallas import tpu_sc as plsc

_B, _N, _M = 4, 16384, 512
_BIG = 3.0e38


def _topk_body(k, n_cols, row_off_stride, a_ref, b_ref, out_ref, dist_s):
    mb = a_ref.shape[2]
    av = a_ref[0]
    bv = b_ref[0]
    a2 = jnp.sum(av * av, axis=0)
    b2 = jnp.sum(bv * bv, axis=0)
    cross = lax.dot_general(
        av, bv, dimension_numbers=(((0,), (0,)), ((), ())),
        preferred_element_type=jnp.float32, precision=lax.Precision.HIGHEST)
    d = jnp.maximum(a2[:, None] + b2[None, :] - 2.0 * cross, 0.0)
    boff = pl.program_id(0) * row_off_stride
    iota = lax.broadcasted_iota(jnp.int32, (mb, n_cols), 1)
    E = 8 if k % 8 == 0 else 4

    def extract(e_base, dd):
        for e in range(E):
            am = jnp.argmin(dd, axis=1).astype(jnp.int32)
            out_ref[0, pl.ds(e_base + e, 1), :] = (am + boff)[None, :]
            dd = jnp.where(iota == am[:, None], _BIG, dd)
        return dd

    dist_s[...] = extract(0, d)

    def body(j, carry):
        dist_s[...] = extract(j * E, dist_s[...])
        return carry

    lax.fori_loop(1, k // E, body, 0)


def _topk(a, b, k, mb):
    B, _, M = a.shape
    N = b.shape[2]
    fn = functools.partial(_topk_body, k, N, N)
    return pl.pallas_call(
        fn,
        grid=(B, M // mb),
        in_specs=[
            pl.BlockSpec((1, 3, mb), lambda bi, mi: (bi, 0, mi)),
            pl.BlockSpec((1, 3, N), lambda bi, mi: (bi, 0, 0)),
        ],
        out_specs=pl.BlockSpec((1, k, mb), lambda bi, mi: (bi, 0, mi)),
        out_shape=jax.ShapeDtypeStruct((B, k, M), jnp.int32),
        scratch_shapes=[pltpu.VMEM((mb, N), jnp.float32)],
    )(a, b)


def _sc_gather(table, idx):
    V, D = table.shape
    Bt = idx.shape[0]
    info = plsc.get_sparse_core_info()
    nc, ns = info.num_cores, info.num_subcores
    nw = nc * ns
    bpw = Bt // nw
    rows = bpw
    while rows * D * 4 > 262144:
        rows //= 2
    nch = bpw // rows
    mesh = plsc.VectorSubcoreMesh(core_axis_name="c", subcore_axis_name="s")

    @functools.partial(
        pl.kernel, mesh=mesh,
        out_type=jax.ShapeDtypeStruct((Bt, D), jnp.float32),
        scratch_types=[
            pltpu.VMEM((rows,), jnp.int32),
            pltpu.VMEM((rows, D), jnp.float32),
            pltpu.SemaphoreType.DMA,
        ],
    )
    def k(tab_hbm, idx_hbm, out_hbm, idx_v, rows_v, sem):
        wid = lax.axis_index("s") * nc + lax.axis_index("c")
        base = wid * bpw
        for c in range(nch):
            off = base + c * rows
            pltpu.sync_copy(idx_hbm.at[pl.ds(off, rows)], idx_v)
            pltpu.async_copy(tab_hbm.at[idx_v], rows_v, sem).wait()
            pltpu.sync_copy(rows_v, out_hbm.at[pl.ds(off, rows)])

    return k(table, idx)


def _mm(x, w, b, prec=lax.Precision.DEFAULT):
    return lax.dot_general(
        x, w, dimension_numbers=(((1,), (0,)), ((), ())),
        preferred_element_type=jnp.float32, precision=prec) + b


def _stage1_body(g_ref, nd_ref, w1, b1, w2, b2, w3, b3, w4, b4, w5, b5,
                 out_ref):
    mb, K, Dp = g_ref.shape[1], g_ref.shape[2], g_ref.shape[3]
    g = g_ref[0]
    nd = nd_ref[0]
    rel = g[..., 0:3] - nd[:, None, :]
    inp = jnp.concatenate([rel, g[..., 3:6]], axis=-1).reshape(mb * K, 6)
    h = jnp.maximum(_mm(inp, w1[...], b1[...]), 0.0)
    h = jnp.maximum(_mm(h, w2[...], b2[...]), 0.0)
    h = jnp.maximum(_mm(h, w3[...], b3[...]), 0.0)
    h3 = h.reshape(mb, K, 64)
    hm = jnp.max(h3, axis=1)
    cat = jnp.concatenate(
        [h3, jnp.broadcast_to(hm[:, None, :], h3.shape)], axis=-1
    ).reshape(mb * K, 128)
    h = jnp.maximum(_mm(cat, w4[...], b4[...]), 0.0)
    h = jnp.maximum(_mm(h, w5[...], b5[...]), 0.0)
    out_ref[0] = jnp.max(h.reshape(mb, K, 128), axis=1)


def _stage1(g, node_t, w1, b1, w2, b2, w3, b3, w4, b4, w5, b5, mb=64):
    B, M, K, Dp = g.shape
    full = lambda s: pl.BlockSpec(s, lambda bi, mi: tuple(0 for _ in s))
    return pl.pallas_call(
        _stage1_body,
        grid=(B, M // mb),
        in_specs=[
            pl.BlockSpec((1, mb, K, Dp), lambda bi, mi: (bi, mi, 0, 0)),
            pl.BlockSpec((1, mb, 3), lambda bi, mi: (bi, mi, 0)),
            full(w1.shape), full(b1.shape), full(w2.shape), full(b2.shape),
            full(w3.shape), full(b3.shape), full(w4.shape), full(b4.shape),
            full(w5.shape), full(b5.shape),
        ],
        out_specs=pl.BlockSpec((1, mb, 128), lambda bi, mi: (bi, mi, 0)),
        out_shape=jax.ShapeDtypeStruct((B, M, 128), jnp.float32),
    )(g, node_t, w1, b1, w2, b2, w3, b3, w4, b4, w5, b5)


def _stage2_body(g_ref, nd_ref, wk1, bk1, wk2, bk2, wk3, bk3, wa1, ba1,
                 wa2, ba2, out_ref):
    mb, K = g_ref.shape[1], g_ref.shape[2]
    g = g_ref[0]
    nd = nd_ref[0]
    rel = g[..., 0:3] - nd[:, None, :]
    inp = jnp.concatenate([rel, g[..., 3:131]], axis=-1).reshape(mb * K, 131)
    h = jnp.maximum(_mm(inp, wk1[...], bk1[...]), 0.0)
    h = jnp.maximum(_mm(h, wk2[...], bk2[...]), 0.0)
    h = jnp.maximum(_mm(h, wk3[...], bk3[...]), 0.0)
    h3 = h.reshape(mb, K, 256)
    hm = jnp.max(h3, axis=1)
    cat = jnp.concatenate(
        [h3, jnp.broadcast_to(hm[:, None, :], h3.shape)], axis=-1
    ).reshape(mb * K, 512)
    h = jnp.maximum(_mm(cat, wa1[...], ba1[...]), 0.0)
    h = jnp.maximum(_mm(h, wa2[...], ba2[...]), 0.0)
    out_ref[0] = jnp.max(h.reshape(mb, K, 512), axis=1)


def _stage2(g, node_t, wk1, bk1, wk2, bk2, wk3, bk3, wa1, ba1, wa2, ba2,
            mb=64):
    B, M, K, Dp = g.shape
    full = lambda s: pl.BlockSpec(s, lambda bi, mi: tuple(0 for _ in s))
    return pl.pallas_call(
        _stage2_body,
        grid=(B, M // mb),
        in_specs=[
            pl.BlockSpec((1, mb, K, Dp), lambda bi, mi: (bi, mi, 0, 0)),
            pl.BlockSpec((1, mb, 3), lambda bi, mi: (bi, mi, 0)),
            full(wk1.shape), full(bk1.shape), full(wk2.shape),
            full(bk2.shape), full(wk3.shape), full(bk3.shape),
            full(wa1.shape), full(ba1.shape), full(wa2.shape),
            full(ba2.shape),
        ],
        out_specs=pl.BlockSpec((1, mb, 512), lambda bi, mi: (bi, mi, 0)),
        out_shape=jax.ShapeDtypeStruct((B, M, 512), jnp.float32),
    )(g, node_t, wk1, bk1, wk2, bk2, wk3, bk3, wa1, ba1, wa2, ba2)


def _final_body(spn_ref, knn_ref, nd_ref, wm1, bm1, wm2, bm2, wm3, bm3,
                kp_ref, sg_ref):
    hi = lax.Precision.HIGHEST
    feat = jnp.concatenate([spn_ref[0], knn_ref[0]], axis=-1)
    y = jnp.maximum(_mm(feat, wm1[...], bm1[...], hi), 0.0)
    pd = jnp.maximum(_mm(y, wm2[...], bm2[...], hi), 0.0)
    ks = _mm(pd, wm3[...], bm3[...], hi)
    kp_ref[0] = ks[:, 0:3] + nd_ref[0]
    s = ks[:, 3:4]
    sg_ref[0] = jnp.maximum(s, 0.0) + jnp.log1p(jnp.exp(-jnp.abs(s))) + 0.001


def _final(spn, knn1, node_t, wm1, bm1, wm2, bm2, wm3, bm3):
    B, M, _ = spn.shape
    full = lambda s: pl.BlockSpec(s, lambda bi: tuple(0 for _ in s))
    return pl.pallas_call(
        _final_body,
        grid=(B,),
        in_specs=[
            pl.BlockSpec((1, M, 128), lambda bi: (bi, 0, 0)),
            pl.BlockSpec((1, M, 512), lambda bi: (bi, 0, 0)),
            pl.BlockSpec((1, M, 3), lambda bi: (bi, 0, 0)),
            full(wm1.shape), full(bm1.shape), full(wm2.shape),
            full(bm2.shape), full(wm3.shape), full(bm3.shape),
        ],
        out_specs=[
            pl.BlockSpec((1, M, 3), lambda bi: (bi, 0, 0)),
            pl.BlockSpec((1, M, 1), lambda bi: (bi, 0, 0)),
        ],
        out_shape=[
            jax.ShapeDtypeStruct((B, M, 3), jnp.float32),
            jax.ShapeDtypeStruct((B, M, 1), jnp.float32),
        ],
    )(spn, knn1, node_t, wm1, bm1, wm2, bm2, wm3, bm3)


def kernel(x, sn, node, W1, b1, W2, b2, W3, b3, W4, b4, W5, b5, Wk1, bk1,
           Wk2, bk2, Wk3, bk3, Wa1, ba1, Wa2, ba2, Wm1, bm1, Wm2, bm2,
           Wm3, bm3):
    B, _, N = x.shape
    M = node.shape[2]
    node_t = jnp.transpose(node, (0, 2, 1))

    idx1 = _topk(node, x, k=64, mb=128)
    tab1 = jnp.transpose(jnp.concatenate([x, sn], axis=1), (0, 2, 1))
    tab1 = jnp.pad(tab1.reshape(B * N, 6), ((0, 0), (0, 122)))
    flat1 = jnp.transpose(idx1, (0, 2, 1)).reshape(B * M * 64)
    g1 = _sc_gather(tab1, flat1).reshape(B, M, 64, 128)
    spn = _stage1(g1, node_t, W1.T, b1[None, :], W2.T, b2[None, :],
                  W3.T, b3[None, :], W4.T, b4[None, :], W5.T, b5[None, :])

    idx2 = _topk(node, node, k=16, mb=128)
    tab2 = jnp.concatenate([node_t.reshape(B * M, 3),
                            spn.reshape(B * M, 128)], axis=1)
    tab2 = jnp.pad(tab2, ((0, 0), (0, 125)))
    flat2 = jnp.transpose(idx2, (0, 2, 1)).reshape(B * M * 16)
    g2 = _sc_gather(tab2, flat2).reshape(B, M, 16, 256)
    knn1 = _stage2(g2, node_t, Wk1.T, bk1[None, :], Wk2.T, bk2[None, :],
                   Wk3.T, bk3[None, :], Wa1.T, ba1[None, :],
                   Wa2.T, ba2[None, :])

    kp, sg = _final(spn, knn1, node_t, Wm1.T, bm1[None, :],
                    Wm2.T, bm2[None, :], Wm3.T, bm3[None, :])
    keypoints = jnp.transpose(kp, (0, 2, 1))
    sigmas = sg.reshape(B, M)
    return node, keypoints, sigmas


# speedup vs baseline: 28.0220x; 1.0000x over previous
import functools

import jax
import jax.numpy as jnp
from jax import lax
from jax.experimental import pallas as pl
from jax.experimental.pallas import tpu as pltpu
from jax.experimental.pallas import tpu_sc as plsc

_B, _N, _M = 4, 16384, 512
_BIG = 3.0e38


def _topk_body(k, n_cols, row_off_stride, a_ref, b_ref, out_ref, dist_s):
    mb = a_ref.shape[2]
    av = a_ref[0]
    bv = b_ref[0]
    a2 = jnp.sum(av * av, axis=0)
    b2 = jnp.sum(bv * bv, axis=0)
    cross = lax.dot_general(
        av, bv, dimension_numbers=(((0,), (0,)), ((), ())),
        preferred_element_type=jnp.float32, precision=lax.Precision.HIGHEST)
    d = jnp.maximum(a2[:, None] + b2[None, :] - 2.0 * cross, 0.0)
    boff = pl.program_id(0) * row_off_stride
    iota = lax.broadcasted_iota(jnp.int32, (mb, n_cols), 1)
    E = 8 if k % 8 == 0 else 4

    def extract(e_base, dd):
        for e in range(E):
            am = jnp.argmin(dd, axis=1).astype(jnp.int32)
            out_ref[0, pl.ds(e_base + e, 1), :] = (am + boff)[None, :]
            dd = jnp.where(iota == am[:, None], _BIG, dd)
        return dd

    dist_s[...] = extract(0, d)

    def body(j, carry):
        dist_s[...] = extract(j * E, dist_s[...])
        return carry

    lax.fori_loop(1, k // E, body, 0)


def _topk(a, b, k, mb):
    B, _, M = a.shape
    N = b.shape[2]
    fn = functools.partial(_topk_body, k, N, N)
    return pl.pallas_call(
        fn,
        grid=(B, M // mb),
        in_specs=[
            pl.BlockSpec((1, 3, mb), lambda bi, mi: (bi, 0, mi)),
            pl.BlockSpec((1, 3, N), lambda bi, mi: (bi, 0, 0)),
        ],
        out_specs=pl.BlockSpec((1, k, mb), lambda bi, mi: (bi, 0, mi)),
        out_shape=jax.ShapeDtypeStruct((B, k, M), jnp.int32),
        scratch_shapes=[pltpu.VMEM((mb, N), jnp.float32)],
        compiler_params=pltpu.CompilerParams(
            dimension_semantics=("parallel", "parallel")),
    )(a, b)


def _sc_gather(table, idx):
    V, D = table.shape
    Bt = idx.shape[0]
    info = plsc.get_sparse_core_info()
    nc, ns = info.num_cores, info.num_subcores
    nw = nc * ns
    bpw = Bt // nw
    rows = bpw
    while rows * D * 4 > 262144:
        rows //= 2
    nch = bpw // rows
    mesh = plsc.VectorSubcoreMesh(core_axis_name="c", subcore_axis_name="s")

    @functools.partial(
        pl.kernel, mesh=mesh,
        out_type=jax.ShapeDtypeStruct((Bt, D), jnp.float32),
        scratch_types=[
            pltpu.VMEM((rows,), jnp.int32),
            pltpu.VMEM((rows, D), jnp.float32),
            pltpu.SemaphoreType.DMA,
        ],
    )
    def k(tab_hbm, idx_hbm, out_hbm, idx_v, rows_v, sem):
        wid = lax.axis_index("s") * nc + lax.axis_index("c")
        base = wid * bpw
        for c in range(nch):
            off = base + c * rows
            pltpu.sync_copy(idx_hbm.at[pl.ds(off, rows)], idx_v)
            pltpu.async_copy(tab_hbm.at[idx_v], rows_v, sem).wait()
            pltpu.sync_copy(rows_v, out_hbm.at[pl.ds(off, rows)])

    return k(table, idx)


def _mm(x, w, b, prec=lax.Precision.DEFAULT):
    return lax.dot_general(
        x, w, dimension_numbers=(((1,), (0,)), ((), ())),
        preferred_element_type=jnp.float32, precision=prec) + b


def _stage1_body(g_ref, nd_ref, w1, b1, w2, b2, w3, b3, w4, b4, w5, b5,
                 out_ref):
    mb, K, Dp = g_ref.shape[1], g_ref.shape[2], g_ref.shape[3]
    g = g_ref[0]
    nd = nd_ref[0]
    rel = g[..., 0:3] - nd[:, None, :]
    inp = jnp.concatenate([rel, g[..., 3:6]], axis=-1).reshape(mb * K, 6)
    h = jnp.maximum(_mm(inp, w1[...], b1[...]), 0.0)
    h = jnp.maximum(_mm(h, w2[...], b2[...]), 0.0)
    h = jnp.maximum(_mm(h, w3[...], b3[...]), 0.0)
    h3 = h.reshape(mb, K, 64)
    hm = jnp.max(h3, axis=1)
    cat = jnp.concatenate(
        [h3, jnp.broadcast_to(hm[:, None, :], h3.shape)], axis=-1
    ).reshape(mb * K, 128)
    h = jnp.maximum(_mm(cat, w4[...], b4[...]), 0.0)
    h = jnp.maximum(_mm(h, w5[...], b5[...]), 0.0)
    out_ref[0] = jnp.max(h.reshape(mb, K, 128), axis=1)


def _stage1(g, node_t, w1, b1, w2, b2, w3, b3, w4, b4, w5, b5, mb=64):
    B, M, K, Dp = g.shape
    full = lambda s: pl.BlockSpec(s, lambda bi, mi: tuple(0 for _ in s))
    return pl.pallas_call(
        _stage1_body,
        grid=(B, M // mb),
        in_specs=[
            pl.BlockSpec((1, mb, K, Dp), lambda bi, mi: (bi, mi, 0, 0)),
            pl.BlockSpec((1, mb, 3), lambda bi, mi: (bi, mi, 0)),
            full(w1.shape), full(b1.shape), full(w2.shape), full(b2.shape),
            full(w3.shape), full(b3.shape), full(w4.shape), full(b4.shape),
            full(w5.shape), full(b5.shape),
        ],
        out_specs=pl.BlockSpec((1, mb, 128), lambda bi, mi: (bi, mi, 0)),
        out_shape=jax.ShapeDtypeStruct((B, M, 128), jnp.float32),
        compiler_params=pltpu.CompilerParams(
            dimension_semantics=("parallel", "parallel")),
    )(g, node_t, w1, b1, w2, b2, w3, b3, w4, b4, w5, b5)


def _stage2_body(g_ref, nd_ref, wk1, bk1, wk2, bk2, wk3, bk3, wa1, ba1,
                 wa2, ba2, out_ref):
    mb, K = g_ref.shape[1], g_ref.shape[2]
    g = g_ref[0]
    nd = nd_ref[0]
    rel = g[..., 0:3] - nd[:, None, :]
    inp = jnp.concatenate([rel, g[..., 3:131]], axis=-1).reshape(mb * K, 131)
    h = jnp.maximum(_mm(inp, wk1[...], bk1[...]), 0.0)
    h = jnp.maximum(_mm(h, wk2[...], bk2[...]), 0.0)
    h = jnp.maximum(_mm(h, wk3[...], bk3[...]), 0.0)
    h3 = h.reshape(mb, K, 256)
    hm = jnp.max(h3, axis=1)
    cat = jnp.concatenate(
        [h3, jnp.broadcast_to(hm[:, None, :], h3.shape)], axis=-1
    ).reshape(mb * K, 512)
    h = jnp.maximum(_mm(cat, wa1[...], ba1[...]), 0.0)
    h = jnp.maximum(_mm(h, wa2[...], ba2[...]), 0.0)
    out_ref[0] = jnp.max(h.reshape(mb, K, 512), axis=1)


def _stage2(g, node_t, wk1, bk1, wk2, bk2, wk3, bk3, wa1, ba1, wa2, ba2,
            mb=64):
    B, M, K, Dp = g.shape
    full = lambda s: pl.BlockSpec(s, lambda bi, mi: tuple(0 for _ in s))
    return pl.pallas_call(
        _stage2_body,
        grid=(B, M // mb),
        in_specs=[
            pl.BlockSpec((1, mb, K, Dp), lambda bi, mi: (bi, mi, 0, 0)),
            pl.BlockSpec((1, mb, 3), lambda bi, mi: (bi, mi, 0)),
            full(wk1.shape), full(bk1.shape), full(wk2.shape),
            full(bk2.shape), full(wk3.shape), full(bk3.shape),
            full(wa1.shape), full(ba1.shape), full(wa2.shape),
            full(ba2.shape),
        ],
        out_specs=pl.BlockSpec((1, mb, 512), lambda bi, mi: (bi, mi, 0)),
        out_shape=jax.ShapeDtypeStruct((B, M, 512), jnp.float32),
        compiler_params=pltpu.CompilerParams(
            dimension_semantics=("parallel", "parallel")),
    )(g, node_t, wk1, bk1, wk2, bk2, wk3, bk3, wa1, ba1, wa2, ba2)


def _final_body(spn_ref, knn_ref, nd_ref, wm1, bm1, wm2, bm2, wm3, bm3,
                kp_ref, sg_ref):
    hi = lax.Precision.HIGHEST
    feat = jnp.concatenate([spn_ref[0], knn_ref[0]], axis=-1)
    y = jnp.maximum(_mm(feat, wm1[...], bm1[...], hi), 0.0)
    pd = jnp.maximum(_mm(y, wm2[...], bm2[...], hi), 0.0)
    ks = _mm(pd, wm3[...], bm3[...], hi)
    kp_ref[0] = ks[:, 0:3] + nd_ref[0]
    s = ks[:, 3:4]
    sg_ref[0] = jnp.maximum(s, 0.0) + jnp.log1p(jnp.exp(-jnp.abs(s))) + 0.001


def _final(spn, knn1, node_t, wm1, bm1, wm2, bm2, wm3, bm3):
    B, M, _ = spn.shape
    full = lambda s: pl.BlockSpec(s, lambda bi: tuple(0 for _ in s))
    return pl.pallas_call(
        _final_body,
        grid=(B,),
        in_specs=[
            pl.BlockSpec((1, M, 128), lambda bi: (bi, 0, 0)),
            pl.BlockSpec((1, M, 512), lambda bi: (bi, 0, 0)),
            pl.BlockSpec((1, M, 3), lambda bi: (bi, 0, 0)),
            full(wm1.shape), full(bm1.shape), full(wm2.shape),
            full(bm2.shape), full(wm3.shape), full(bm3.shape),
        ],
        out_specs=[
            pl.BlockSpec((1, M, 3), lambda bi: (bi, 0, 0)),
            pl.BlockSpec((1, M, 1), lambda bi: (bi, 0, 0)),
        ],
        out_shape=[
            jax.ShapeDtypeStruct((B, M, 3), jnp.float32),
            jax.ShapeDtypeStruct((B, M, 1), jnp.float32),
        ],
    )(spn, knn1, node_t, wm1, bm1, wm2, bm2, wm3, bm3)


def kernel(x, sn, node, W1, b1, W2, b2, W3, b3, W4, b4, W5, b5, Wk1, bk1,
           Wk2, bk2, Wk3, bk3, Wa1, ba1, Wa2, ba2, Wm1, bm1, Wm2, bm2,
           Wm3, bm3):
    B, _, N = x.shape
    M = node.shape[2]
    node_t = jnp.transpose(node, (0, 2, 1))

    idx1 = _topk(node, x, k=64, mb=128)
    tab1 = jnp.transpose(jnp.concatenate([x, sn], axis=1), (0, 2, 1))
    tab1 = jnp.pad(tab1.reshape(B * N, 6), ((0, 0), (0, 122)))
    flat1 = jnp.transpose(idx1, (0, 2, 1)).reshape(B * M * 64)
    g1 = _sc_gather(tab1, flat1).reshape(B, M, 64, 128)
    spn = _stage1(g1, node_t, W1.T, b1[None, :], W2.T, b2[None, :],
                  W3.T, b3[None, :], W4.T, b4[None, :], W5.T, b5[None, :])

    idx2 = _topk(node, node, k=16, mb=128)
    tab2 = jnp.concatenate([node_t.reshape(B * M, 3),
                            spn.reshape(B * M, 128)], axis=1)
    tab2 = jnp.pad(tab2, ((0, 0), (0, 125)))
    flat2 = jnp.transpose(idx2, (0, 2, 1)).reshape(B * M * 16)
    g2 = _sc_gather(tab2, flat2).reshape(B, M, 16, 256)
    knn1 = _stage2(g2, node_t, Wk1.T, bk1[None, :], Wk2.T, bk2[None, :],
                   Wk3.T, bk3[None, :], Wa1.T, ba1[None, :],
                   Wa2.T, ba2[None, :])

    kp, sg = _final(spn, knn1, node_t, Wm1.T, bm1[None, :],
                    Wm2.T, bm2[None, :], Wm3.T, bm3[None, :])
    keypoints = jnp.transpose(kp, (0, 2, 1))
    sigmas = sg.reshape(B, M)
    return node, keypoints, sigmas
